# baseline (speedup 1.0000x reference)
import functools

import jax
import jax.numpy as jnp
import numpy as np
from jax import lax
from jax.experimental import pallas as pl
from jax.experimental.pallas import tpu as pltpu
from jax.experimental.pallas import tpu_sc as plsc

N = 9984
F = 128
G = 78
E = 319488
HC = 128
HID = 256
NCLS = 2
IN_DIM = F * (F - 1) // 2
BNC = float(1.0 / np.sqrt(1.0 + 1e-5))

NW = 32
EPW = E // NW
EV = EPW // 16
NV = N // 16


def _deg_partials_body(edge_hbm, out_hbm, idx_v, hist_v):
    c = lax.axis_index("c")
    s = lax.axis_index("s")
    w = s * 2 + c

    pltpu.sync_copy(edge_hbm.at[1, pl.ds(w * EPW, EPW)], idx_v)

    zeros = jnp.zeros((16,), jnp.float32)

    def zero_body(i, carry):
        for u in range(8):
            hist_v[pl.ds(i * 128 + u * 16, 16)] = zeros
        return carry

    lax.fori_loop(0, NV // 8, zero_body, 0)

    def hist_body(i, carry):
        for u in range(4):
            idx = idx_v[pl.ds(i * 64 + u * 16, 16)]
            cnt, last = plsc.scan_count(idx)
            plsc.addupdate_scatter(hist_v, [idx], cnt.astype(jnp.float32),
                                   mask=last)
        return carry

    lax.fori_loop(0, EV // 4, hist_body, 0)

    pltpu.sync_copy(hist_v, out_hbm.at[w])


@functools.cache
def _deg_partials():
    return pl.kernel(
        _deg_partials_body,
        out_type=jax.ShapeDtypeStruct((NW, N), jnp.float32),
        mesh=plsc.VectorSubcoreMesh(core_axis_name="c", subcore_axis_name="s"),
        scratch_types=[
            pltpu.VMEM((EPW,), jnp.int32),
            pltpu.VMEM((N,), jnp.float32),
        ],
        compiler_params=pltpu.CompilerParams(needs_layout_passes=False),
    )


_OFF = [i * (F - 1) - i * (i - 1) // 2 for i in range(F)]
WTPAD = 8192
BNPAD = 8192


def _triu_mm_body(xf_ref, w1_ref, bnp_ref, bnb_ref, bnhb_ref,
                  vx_ref, bias_ref, w1b_ref, w1at_ref, wf_ref):
    w1at_ref[0:8, :] = jnp.zeros((8, HID), jnp.bfloat16)
    w1at_ref[8 + IN_DIM:, :] = jnp.zeros((WTPAD - 8 - IN_DIM, HID),
                                         jnp.bfloat16)
    w1at_ref[8:8 + IN_DIM, :] = jnp.transpose(w1_ref[:, :IN_DIM])

    for i in range(F):
        s = 8 + _OFF[i] - (i + 1)
        wf_ref[pl.ds(i * F, F), :] = w1at_ref[pl.ds(s, F), :]

    jj = lax.broadcasted_iota(jnp.int32, (1, F), 1)
    parts = []
    for i in range(F):
        s = 8 + _OFF[i] - (i + 1)
        parts.append(jnp.where(jj > i, bnp_ref[:, pl.ds(s, F)], 0.0))
    bnf = jnp.concatenate(parts, axis=1)

    xq = (xf_ref[...].astype(jnp.float32) * bnf).astype(jnp.bfloat16)
    nd = (((1,), (1,)), ((), ()))
    vx_ref[...] = jnp.dot(xq, wf_ref[...],
                          preferred_element_type=jnp.float32)
    bias_ref[...] = (
        lax.dot_general(bnb_ref[...].astype(jnp.bfloat16),
                        w1_ref[:, :IN_DIM], nd,
                        preferred_element_type=jnp.float32)
        + lax.dot_general(bnhb_ref[...].astype(jnp.bfloat16),
                          w1_ref[:, IN_DIM:], nd,
                          preferred_element_type=jnp.float32))
    w1b_ref[...] = w1_ref[:, IN_DIM:]


def _triu_mm(xf, w1, bnp, bnb, bnhb):
    return pl.pallas_call(
        _triu_mm_body,
        out_shape=(jax.ShapeDtypeStruct((G, HID), jnp.float32),
                   jax.ShapeDtypeStruct((1, HID), jnp.float32),
                   jax.ShapeDtypeStruct((HID, 2 * HC), jnp.bfloat16)),
        scratch_shapes=[pltpu.VMEM((WTPAD, HID), jnp.bfloat16),
                        pltpu.VMEM((F * F, HID), jnp.bfloat16)],
    )(xf, w1, bnp, bnb, bnhb)


def _conv_pool_body(x_ref, dp_ref, w0t_ref, b0_ref, w1t_ref, b1_ref, h_ref):
    degcol = (jnp.sum(jnp.transpose(dp_ref[...]), axis=1, keepdims=True)
              + 1.0)
    a = jnp.dot(x_ref[...], w0t_ref[...],
                preferred_element_type=jnp.float32) + b0_ref[...]
    x1 = degcol * a
    bm = jnp.dot(x1.astype(jnp.bfloat16), w1t_ref[...],
                 preferred_element_type=jnp.float32) + b1_ref[...]
    x2 = degcol * bm
    h1 = jnp.sum(x1.reshape(G, F, HC), axis=1) * (1.0 / F)
    h2 = jnp.sum(x2.reshape(G, F, HC), axis=1) * (1.0 / F)
    h_ref[...] = jnp.concatenate([h1, h2], axis=1)


def _conv_pool(x, deg_parts, w0t, b0, w1t, b1):
    return pl.pallas_call(
        _conv_pool_body,
        out_shape=jax.ShapeDtypeStruct((G, 2 * HC), jnp.float32),
    )(x, deg_parts, w0t, b0, w1t, b1)


def _mlp_tail_body(vx_ref, bias_ref, h_ref, bnh_ref, w1b_ref, g1_ref,
                   b1r_ref, be1_ref, w2t_ref, beta2_ref, w3t_ref, beta3_ref,
                   w4t_ref, b4_ref, out_ref):
    hq = (h_ref[...] * bnh_ref[...]).astype(jnp.bfloat16)
    nd = (((1,), (1,)), ((), ()))
    vh = lax.dot_general(hq, w1b_ref[...], nd,
                         preferred_element_type=jnp.float32)
    beta1 = g1_ref[...] * (b1r_ref[...] + bias_ref[...]) + be1_ref[...]
    v = (vx_ref[...] + vh) * g1_ref[...] + beta1
    z1 = jnp.maximum(v, 0.0).astype(jnp.bfloat16)
    z2 = jnp.maximum(
        jnp.dot(z1, w2t_ref[...], preferred_element_type=jnp.float32)
        + beta2_ref[...], 0.0).astype(jnp.bfloat16)
    z3 = jnp.maximum(
        jnp.dot(z2, w3t_ref[...], preferred_element_type=jnp.float32)
        + beta3_ref[...], 0.0).astype(jnp.bfloat16)
    out_ref[...] = (jnp.dot(z3, w4t_ref[...],
                            preferred_element_type=jnp.float32)
                    + b4_ref[...])


def _mlp_tail(vx, bias, hpre, bnh, w1b, g1r, b1r, be1, w2t, beta2, w3t,
              beta3, w4t, b4):
    return pl.pallas_call(
        _mlp_tail_body,
        out_shape=jax.ShapeDtypeStruct((G, HID // 2), jnp.float32),
    )(vx, bias, hpre, bnh, w1b, g1r, b1r, be1, w2t, beta2, w3t, beta3,
      w4t, b4)


def kernel(x, edge_index, batch, params):
    deg_parts = _deg_partials()(edge_index)

    bf = jnp.bfloat16
    c = BNC
    convs = params['convs']
    x_bf = x.astype(bf)
    w0t = convs[0]['Wdst'].T.astype(bf)
    b0 = convs[0]['bdst'].reshape(1, HC)
    w1t = convs[1]['Wdst'].T.astype(bf)
    b1 = convs[1]['bdst'].reshape(1, HC)

    w1_bf = params['W1'].astype(bf)
    bnp = jnp.zeros((1, BNPAD), jnp.float32).at[0, 8:8 + IN_DIM].set(
        c * params['bn_g'])
    bnb = params['bn_b'].reshape(1, IN_DIM)
    bnhb = params['bnh_b'].reshape(1, 2 * HC)
    vx, bias, w1b = _triu_mm(x_bf.reshape(G, F * F), w1_bf, bnp, bnb, bnhb)

    hpre = _conv_pool(x_bf, deg_parts, w0t, b0, w1t, b1)

    bnh = (c * params['bnh_g']).reshape(1, HID)
    g1r = (c * params['g1']).reshape(1, HID)
    b1r = params['b1'].reshape(1, HID)
    be1 = params['be1'].reshape(1, HID)
    w2t = (params['W2'].T * (c * params['g2'])[None, :]).astype(bf)
    beta2 = (c * params['g2'] * params['b2'] + params['be2']).reshape(1, HID // 2)
    w3t = (params['W3'].T * (c * params['g3'])[None, :]).astype(bf)
    beta3 = (c * params['g3'] * params['b3'] + params['be3']).reshape(1, HID // 2)
    w4t = jnp.zeros((HID // 2, HID // 2), bf).at[:, :NCLS].set(
        params['W4'].T.astype(bf))
    b4 = jnp.zeros((1, HID // 2), jnp.float32).at[0, :NCLS].set(params['b4'])

    out_full = _mlp_tail(vx, bias, hpre, bnh, w1b, g1r, b1r, be1,
                         w2t, beta2, w3t, beta3, w4t, b4)
    return out_full[:, :NCLS]


# speedup vs baseline: 1.0444x; 1.0444x over previous
import functools

import jax
import jax.numpy as jnp
import numpy as np
from jax import lax
from jax.experimental import pallas as pl
from jax.experimental.pallas import tpu as pltpu
from jax.experimental.pallas import tpu_sc as plsc

N = 9984
F = 128
G = 78
E = 319488
HC = 128
HID = 256
NCLS = 2
IN_DIM = F * (F - 1) // 2
BNC = float(1.0 / np.sqrt(1.0 + 1e-5))

NW = 32
EPW = E // NW
EV = EPW // 16
NV = N // 16


def _deg_partials_body(edge_hbm, out_hbm, idx_v, hist_v):
    c = lax.axis_index("c")
    s = lax.axis_index("s")
    w = s * 2 + c

    pltpu.sync_copy(edge_hbm.at[1, pl.ds(w * EPW, EPW)], idx_v)

    zeros = jnp.zeros((16,), jnp.float32)

    def zero_body(i, carry):
        for u in range(8):
            hist_v[pl.ds(i * 128 + u * 16, 16)] = zeros
        return carry

    lax.fori_loop(0, NV // 8, zero_body, 0)

    def hist_body(i, carry):
        for u in range(4):
            idx = idx_v[pl.ds(i * 64 + u * 16, 16)]
            cnt, last = plsc.scan_count(idx)
            plsc.addupdate_scatter(hist_v, [idx], cnt.astype(jnp.float32),
                                   mask=last)
        return carry

    lax.fori_loop(0, EV // 4, hist_body, 0)

    pltpu.sync_copy(hist_v, out_hbm.at[w])


@functools.cache
def _deg_partials():
    return pl.kernel(
        _deg_partials_body,
        out_type=jax.ShapeDtypeStruct((NW, N), jnp.float32),
        mesh=plsc.VectorSubcoreMesh(core_axis_name="c", subcore_axis_name="s"),
        scratch_types=[
            pltpu.VMEM((EPW,), jnp.int32),
            pltpu.VMEM((N,), jnp.float32),
        ],
        compiler_params=pltpu.CompilerParams(needs_layout_passes=False),
    )


_OFF = [i * (F - 1) - i * (i - 1) // 2 for i in range(F)]
WTPAD = 8192
BNPAD = 8192


def _triu_mm_body(xf_ref, w1_ref, bnp_ref, bnb_ref, bnhb_ref,
                  vx_ref, bias_ref, w1b_ref, w1at_ref, wf_ref):
    w1a_bf = w1_ref[:, :IN_DIM].astype(jnp.bfloat16)
    w1b_bf = w1_ref[:, IN_DIM:].astype(jnp.bfloat16)
    w1at_ref[0:8, :] = jnp.zeros((8, HID), jnp.bfloat16)
    w1at_ref[8 + IN_DIM:, :] = jnp.zeros((WTPAD - 8 - IN_DIM, HID),
                                         jnp.bfloat16)
    w1at_ref[8:8 + IN_DIM, :] = jnp.transpose(w1a_bf)

    for i in range(F):
        s = 8 + _OFF[i] - (i + 1)
        wf_ref[pl.ds(i * F, F), :] = w1at_ref[pl.ds(s, F), :]

    jj = lax.broadcasted_iota(jnp.int32, (1, F), 1)
    parts = []
    for i in range(F):
        s = 8 + _OFF[i] - (i + 1)
        parts.append(jnp.where(jj > i, bnp_ref[:, pl.ds(s, F)], 0.0))
    bnf = jnp.concatenate(parts, axis=1)

    xq = (xf_ref[...] * bnf).astype(jnp.bfloat16)
    nd = (((1,), (1,)), ((), ()))
    vx_ref[...] = jnp.dot(xq, wf_ref[...],
                          preferred_element_type=jnp.float32)
    bias_ref[...] = (
        lax.dot_general(bnb_ref[...].astype(jnp.bfloat16), w1a_bf, nd,
                        preferred_element_type=jnp.float32)
        + lax.dot_general(bnhb_ref[...].astype(jnp.bfloat16), w1b_bf, nd,
                          preferred_element_type=jnp.float32))
    w1b_ref[...] = w1b_bf


def _triu_mm(xf, w1, bnp, bnb, bnhb):
    return pl.pallas_call(
        _triu_mm_body,
        out_shape=(jax.ShapeDtypeStruct((G, HID), jnp.float32),
                   jax.ShapeDtypeStruct((1, HID), jnp.float32),
                   jax.ShapeDtypeStruct((HID, 2 * HC), jnp.bfloat16)),
        scratch_shapes=[pltpu.VMEM((WTPAD, HID), jnp.bfloat16),
                        pltpu.VMEM((F * F, HID), jnp.bfloat16)],
    )(xf, w1, bnp, bnb, bnhb)


def _conv_pool_body(x_ref, dp_ref, w0t_ref, b0_ref, w1t_ref, b1_ref, h_ref):
    degcol = (jnp.sum(jnp.transpose(dp_ref[...]), axis=1, keepdims=True)
              + 1.0)
    a = jnp.dot(x_ref[...].astype(jnp.bfloat16), w0t_ref[...],
                preferred_element_type=jnp.float32) + b0_ref[...]
    x1 = degcol * a
    bm = jnp.dot(x1.astype(jnp.bfloat16), w1t_ref[...],
                 preferred_element_type=jnp.float32) + b1_ref[...]
    x2 = degcol * bm
    h1 = jnp.sum(x1.reshape(G, F, HC), axis=1) * (1.0 / F)
    h2 = jnp.sum(x2.reshape(G, F, HC), axis=1) * (1.0 / F)
    h_ref[...] = jnp.concatenate([h1, h2], axis=1)


def _conv_pool(x, deg_parts, w0t, b0, w1t, b1):
    return pl.pallas_call(
        _conv_pool_body,
        out_shape=jax.ShapeDtypeStruct((G, 2 * HC), jnp.float32),
    )(x, deg_parts, w0t, b0, w1t, b1)


def _mlp_tail_body(vx_ref, bias_ref, h_ref, bnh_ref, w1b_ref, g1_ref,
                   b1r_ref, be1_ref, w2t_ref, beta2_ref, w3t_ref, beta3_ref,
                   w4_ref, b4_ref, out_ref):
    hq = (h_ref[...] * bnh_ref[...]).astype(jnp.bfloat16)
    nd = (((1,), (1,)), ((), ()))
    vh = lax.dot_general(hq, w1b_ref[...], nd,
                         preferred_element_type=jnp.float32)
    beta1 = g1_ref[...] * (b1r_ref[...] + bias_ref[...]) + be1_ref[...]
    v = (vx_ref[...] + vh) * g1_ref[...] + beta1
    z1 = jnp.maximum(v, 0.0).astype(jnp.bfloat16)
    z2 = jnp.maximum(
        jnp.dot(z1, w2t_ref[...], preferred_element_type=jnp.float32)
        + beta2_ref[...], 0.0).astype(jnp.bfloat16)
    z3 = jnp.maximum(
        jnp.dot(z2, w3t_ref[...], preferred_element_type=jnp.float32)
        + beta3_ref[...], 0.0).astype(jnp.bfloat16)
    out_ref[...] = (lax.dot_general(z3, w4_ref[...].astype(jnp.bfloat16),
                                    nd, preferred_element_type=jnp.float32)
                    + b4_ref[...])


def _mlp_tail(vx, bias, hpre, bnh, w1b, g1r, b1r, be1, w2t, beta2, w3t,
              beta3, w4, b4):
    return pl.pallas_call(
        _mlp_tail_body,
        out_shape=jax.ShapeDtypeStruct((G, NCLS), jnp.float32),
    )(vx, bias, hpre, bnh, w1b, g1r, b1r, be1, w2t, beta2, w3t, beta3,
      w4, b4)


def kernel(x, edge_index, batch, params):
    deg_parts = _deg_partials()(edge_index)

    bf = jnp.bfloat16
    c = BNC
    convs = params['convs']
    w0t = convs[0]['Wdst'].T.astype(bf)
    b0 = convs[0]['bdst'].reshape(1, HC)
    w1t = convs[1]['Wdst'].T.astype(bf)
    b1 = convs[1]['bdst'].reshape(1, HC)

    bnp = jnp.zeros((1, BNPAD), jnp.float32).at[0, 8:8 + IN_DIM].set(
        c * params['bn_g'])
    bnb = params['bn_b'].reshape(1, IN_DIM)
    bnhb = params['bnh_b'].reshape(1, 2 * HC)
    vx, bias, w1b = _triu_mm(x.reshape(G, F * F), params['W1'], bnp, bnb,
                             bnhb)

    hpre = _conv_pool(x, deg_parts, w0t, b0, w1t, b1)

    bnh = (c * params['bnh_g']).reshape(1, HID)
    g1r = (c * params['g1']).reshape(1, HID)
    b1r = params['b1'].reshape(1, HID)
    be1 = params['be1'].reshape(1, HID)
    w2t = (params['W2'].T * (c * params['g2'])[None, :]).astype(bf)
    beta2 = (c * params['g2'] * params['b2'] + params['be2']).reshape(1, HID // 2)
    w3t = (params['W3'].T * (c * params['g3'])[None, :]).astype(bf)
    beta3 = (c * params['g3'] * params['b3'] + params['be3']).reshape(1, HID // 2)
    b4 = params['b4'].reshape(1, NCLS)

    return _mlp_tail(vx, bias, hpre, bnh, w1b, g1r, b1r, be1,
                     w2t, beta2, w3t, beta3, params['W4'], b4)


# speedup vs baseline: 1.2484x; 1.1953x over previous
import functools

import jax
import jax.numpy as jnp
import numpy as np
from jax import lax
from jax.experimental import pallas as pl
from jax.experimental.pallas import tpu as pltpu
from jax.experimental.pallas import tpu_sc as plsc

N = 9984
F = 128
G = 78
E = 319488
HC = 128
HID = 256
NCLS = 2
IN_DIM = F * (F - 1) // 2
BNC = float(1.0 / np.sqrt(1.0 + 1e-5))

NW = 32
EPW = E // NW
EV = EPW // 16
NV = N // 16


def _deg_partials_body(edge_hbm, out_hbm, idx_v, hist_v):
    c = lax.axis_index("c")
    s = lax.axis_index("s")
    w = s * 2 + c

    pltpu.sync_copy(edge_hbm.at[1, pl.ds(w * EPW, EPW)], idx_v)

    zeros = jnp.zeros((16,), jnp.float32)

    def zero_body(i, carry):
        for u in range(8):
            hist_v[pl.ds(i * 128 + u * 16, 16)] = zeros
        return carry

    lax.fori_loop(0, NV // 8, zero_body, 0)

    def hist_body(i, carry):
        for u in range(4):
            idx = idx_v[pl.ds(i * 64 + u * 16, 16)]
            cnt, last = plsc.scan_count(idx)
            plsc.addupdate_scatter(hist_v, [idx], cnt.astype(jnp.float32),
                                   mask=last)
        return carry

    lax.fori_loop(0, EV // 4, hist_body, 0)

    pltpu.sync_copy(hist_v, out_hbm.at[w])


@functools.cache
def _deg_partials():
    return pl.kernel(
        _deg_partials_body,
        out_type=jax.ShapeDtypeStruct((NW, N), jnp.float32),
        mesh=plsc.VectorSubcoreMesh(core_axis_name="c", subcore_axis_name="s"),
        scratch_types=[
            pltpu.VMEM((EPW,), jnp.int32),
            pltpu.VMEM((N,), jnp.float32),
        ],
        compiler_params=pltpu.CompilerParams(needs_layout_passes=False),
    )


_OFF = [i * (F - 1) - i * (i - 1) // 2 for i in range(F)]
WTPAD = 8192
BNPAD = 8192


def _triu_mm_body(xf_ref, w1t_ref, bnp_ref, bnb_ref, bnhb_ref,
                  vx_ref, bias_ref, w1bt_ref, w1bf_ref, wf_ref):
    w1bf_ref[...] = w1t_ref[...].astype(jnp.bfloat16)

    for i in range(F):
        if i == 0:
            wf_ref[0:F, :] = w1bf_ref[0:F, :]
            wf_ref[1:F, :] = w1bf_ref[0:F - 1, :]
        else:
            s = _OFF[i] - (i + 1)
            wf_ref[pl.ds(i * F, F), :] = w1bf_ref[pl.ds(s, F), :]

    jj = lax.broadcasted_iota(jnp.int32, (1, F), 1)
    parts = []
    for i in range(F):
        s = 8 + _OFF[i] - (i + 1)
        parts.append(jnp.where(jj > i, bnp_ref[:, pl.ds(s, F)], 0.0))
    bnf = jnp.concatenate(parts, axis=1)

    xq = (xf_ref[...] * bnf).astype(jnp.bfloat16)
    vx_ref[...] = jnp.dot(xq, wf_ref[...],
                          preferred_element_type=jnp.float32)
    bias_ref[...] = (
        jnp.dot(bnb_ref[...].astype(jnp.bfloat16), w1bf_ref[0:IN_DIM, :],
                preferred_element_type=jnp.float32)
        + jnp.dot(bnhb_ref[...].astype(jnp.bfloat16),
                  w1bf_ref[IN_DIM:, :],
                  preferred_element_type=jnp.float32))
    w1bt_ref[...] = w1bf_ref[IN_DIM:, :]


def _triu_mm(xf, w1t, bnp, bnb, bnhb):
    return pl.pallas_call(
        _triu_mm_body,
        out_shape=(jax.ShapeDtypeStruct((G, HID), jnp.float32),
                   jax.ShapeDtypeStruct((1, HID), jnp.float32),
                   jax.ShapeDtypeStruct((2 * HC, HID), jnp.bfloat16)),
        scratch_shapes=[pltpu.VMEM((IN_DIM + 2 * HC, HID), jnp.bfloat16),
                        pltpu.VMEM((F * F, HID), jnp.bfloat16)],
    )(xf, w1t, bnp, bnb, bnhb)


def _conv_pool_body(x_ref, dp_ref, w0t_ref, b0_ref, w1t_ref, b1_ref, h_ref):
    degcol = (jnp.sum(jnp.transpose(dp_ref[...]), axis=1, keepdims=True)
              + 1.0)
    a = jnp.dot(x_ref[...].astype(jnp.bfloat16), w0t_ref[...],
                preferred_element_type=jnp.float32) + b0_ref[...]
    x1 = degcol * a
    bm = jnp.dot(x1.astype(jnp.bfloat16), w1t_ref[...],
                 preferred_element_type=jnp.float32) + b1_ref[...]
    x2 = degcol * bm
    h1 = jnp.sum(x1.reshape(G, F, HC), axis=1) * (1.0 / F)
    h2 = jnp.sum(x2.reshape(G, F, HC), axis=1) * (1.0 / F)
    h_ref[...] = jnp.concatenate([h1, h2], axis=1)


def _conv_pool(x, deg_parts, w0t, b0, w1t, b1):
    return pl.pallas_call(
        _conv_pool_body,
        out_shape=jax.ShapeDtypeStruct((G, 2 * HC), jnp.float32),
    )(x, deg_parts, w0t, b0, w1t, b1)


def _mlp_tail_body(vx_ref, bias_ref, h_ref, bnh_ref, w1b_ref, g1_ref,
                   b1r_ref, be1_ref, w2t_ref, beta2_ref, w3t_ref, beta3_ref,
                   w4_ref, b4_ref, out_ref):
    hq = (h_ref[...] * bnh_ref[...]).astype(jnp.bfloat16)
    vh = jnp.dot(hq, w1b_ref[...], preferred_element_type=jnp.float32)
    beta1 = g1_ref[...] * (b1r_ref[...] + bias_ref[...]) + be1_ref[...]
    v = (vx_ref[...] + vh) * g1_ref[...] + beta1
    z1 = jnp.maximum(v, 0.0).astype(jnp.bfloat16)
    z2 = jnp.maximum(
        jnp.dot(z1, w2t_ref[...], preferred_element_type=jnp.float32)
        + beta2_ref[...], 0.0).astype(jnp.bfloat16)
    z3 = jnp.maximum(
        jnp.dot(z2, w3t_ref[...], preferred_element_type=jnp.float32)
        + beta3_ref[...], 0.0).astype(jnp.bfloat16)
    nd = (((1,), (1,)), ((), ()))
    out_ref[...] = (lax.dot_general(w4_ref[...].astype(jnp.bfloat16), z3,
                                    nd, preferred_element_type=jnp.float32)
                    + b4_ref[...])


def _mlp_tail(vx, bias, hpre, bnh, w1b, g1r, b1r, be1, w2t, beta2, w3t,
              beta3, w4, b4):
    return pl.pallas_call(
        _mlp_tail_body,
        out_shape=jax.ShapeDtypeStruct((NCLS, G), jnp.float32),
    )(vx, bias, hpre, bnh, w1b, g1r, b1r, be1, w2t, beta2, w3t, beta3,
      w4, b4)


def kernel(x, edge_index, batch, params):
    deg_parts = _deg_partials()(edge_index)

    bf = jnp.bfloat16
    c = BNC
    convs = params['convs']
    w0t = convs[0]['Wdst'].T.astype(bf)
    b0 = convs[0]['bdst'].reshape(1, HC)
    w1t = convs[1]['Wdst'].T.astype(bf)
    b1 = convs[1]['bdst'].reshape(1, HC)

    bnp = jnp.zeros((1, BNPAD), jnp.float32).at[0, 8:8 + IN_DIM].set(
        c * params['bn_g'])
    bnb = params['bn_b'].reshape(1, IN_DIM)
    bnhb = params['bnh_b'].reshape(1, 2 * HC)
    vx, bias, w1b = _triu_mm(x.reshape(G, F * F), params['W1'].T, bnp, bnb,
                             bnhb)

    hpre = _conv_pool(x, deg_parts, w0t, b0, w1t, b1)

    bnh = (c * params['bnh_g']).reshape(1, HID)
    g1r = (c * params['g1']).reshape(1, HID)
    b1r = params['b1'].reshape(1, HID)
    be1 = params['be1'].reshape(1, HID)
    w2t = (params['W2'].T * (c * params['g2'])[None, :]).astype(bf)
    beta2 = (c * params['g2'] * params['b2'] + params['be2']).reshape(1, HID // 2)
    w3t = (params['W3'].T * (c * params['g3'])[None, :]).astype(bf)
    beta3 = (c * params['g3'] * params['b3'] + params['be3']).reshape(1, HID // 2)
    b4 = params['b4'].reshape(NCLS, 1)

    return _mlp_tail(vx, bias, hpre, bnh, w1b, g1r, b1r, be1,
                     w2t, beta2, w3t, beta3, params['W4'], b4).T
